# Initial kernel scaffold; baseline (speedup 1.0000x reference)
import jax
import jax.numpy as jnp
from jax.experimental import pallas as pl


def kernel(time_features, month_table, week_table, holiday_table, W_week, b_week, W_holiday, b_holiday, pe):
    raise NotImplementedError("write your pallas kernel here")


# speedup vs baseline: 9.3266x; 9.3266x over previous
import functools

import jax
import jax.numpy as jnp
from jax import lax
from jax.experimental import pallas as pl
from jax.experimental.pallas import tpu as pltpu
from jax.experimental.pallas import tpu_sc as plsc

_B, _L, _H = 4096, 200, 128
_NCOMBO = 12 * 7 * 2
_NC, _NS = 2, 16
_NW = _NC * _NS
_TOK = _B * _L
_ROWS_W = _TOK // _NW
_CH = 128
_NCHUNK = _ROWS_W // _CH
_LBLK = 8


def _table_body(month_ref, week_ref, hol_ref, ww_ref, bw_ref, wh_ref, bh_ref,
                pe_ref, out_ref, comb_ref):
    @pl.when(pl.program_id(0) == 0)
    def _():
        wproj = jnp.dot(week_ref[...], ww_ref[...],
                        preferred_element_type=jnp.float32) + bw_ref[...]
        hproj = jnp.dot(hol_ref[...], wh_ref[...],
                        preferred_element_type=jnp.float32) + bh_ref[...]
        r_m = lax.broadcasted_iota(jnp.int32, (_NCOMBO, 12), 0) // 14
        c_m = lax.broadcasted_iota(jnp.int32, (_NCOMBO, 12), 1)
        s_m = (r_m == c_m).astype(jnp.float32)
        r_w = (lax.broadcasted_iota(jnp.int32, (_NCOMBO, 7), 0) % 14) // 2
        c_w = lax.broadcasted_iota(jnp.int32, (_NCOMBO, 7), 1)
        s_w = (r_w == c_w).astype(jnp.float32)
        r_h = lax.broadcasted_iota(jnp.int32, (_NCOMBO, 2), 0) % 2
        c_h = lax.broadcasted_iota(jnp.int32, (_NCOMBO, 2), 1)
        s_h = (r_h == c_h).astype(jnp.float32)
        comb_ref[...] = (
            jnp.dot(s_m, month_ref[...], preferred_element_type=jnp.float32)
            + jnp.dot(s_w, wproj, preferred_element_type=jnp.float32)
            + jnp.dot(s_h, hproj, preferred_element_type=jnp.float32))

    out_ref[...] = comb_ref[...][None, :, :] + pe_ref[...][:, None, :]


def _build_table(month, week, hol, ww, bw2, wh, bh2, pe2):
    grid = _L // _LBLK
    const = lambda blk: pl.BlockSpec(blk, lambda i: tuple(0 for _ in blk))
    return pl.pallas_call(
        _table_body,
        grid=(grid,),
        in_specs=[
            const((12, _H)),
            const((7, _H // 2)),
            const((2, _H // 4)),
            const((_H // 2, _H)),
            const((1, _H)),
            const((_H // 4, _H)),
            const((1, _H)),
            pl.BlockSpec((_LBLK, _H), lambda i: (i, 0)),
        ],
        out_specs=pl.BlockSpec((_LBLK, _NCOMBO, _H), lambda i: (i, 0, 0)),
        out_shape=jax.ShapeDtypeStruct((_L, _NCOMBO, _H), jnp.float32),
        scratch_shapes=[pltpu.VMEM((_NCOMBO, _H), jnp.float32)],
    )(month, week, hol, ww, bw2, wh, bh2, pe2)


def _idx_body(m_ref, w_ref, h_ref, out_ref):
    rows, cols = m_ref.shape
    row = lax.broadcasted_iota(jnp.int32, (rows, cols), 0)
    col = lax.broadcasted_iota(jnp.int32, (rows, cols), 1)
    flat = (pl.program_id(0) * rows + row) * cols + col
    l = flat % _L
    out_ref[...] = (l * _NCOMBO + (m_ref[...] - 1) * 14
                    + w_ref[...] * 2 + h_ref[...])


def _build_idx(m2, w2, h2):
    rows = m2.shape[0]
    rblk = 800
    grid = rows // rblk
    spec = pl.BlockSpec((rblk, 128), lambda i: (i, 0))
    return pl.pallas_call(
        _idx_body,
        grid=(grid,),
        in_specs=[spec, spec, spec],
        out_specs=spec,
        out_shape=jax.ShapeDtypeStruct((rows, 128), jnp.int32),
    )(m2, w2, h2)


def _gather_body(t_hbm, g_hbm, out_hbm, idx_v, rows_v, sem):
    wid = lax.axis_index("s") * _NC + lax.axis_index("c")
    pltpu.sync_copy(g_hbm.at[pl.ds(wid * _NCHUNK, _NCHUNK)], idx_v)
    base = wid * _ROWS_W

    def body(i, carry):
        pltpu.async_copy(t_hbm.at[idx_v.at[i]], rows_v, sem).wait()
        pltpu.sync_copy(rows_v, out_hbm.at[pl.ds(base + i * _CH, _CH)])
        return carry

    lax.fori_loop(0, _NCHUNK, body, 0)


def _sc_gather(t2, gidx2):
    mesh = plsc.VectorSubcoreMesh(core_axis_name="c", subcore_axis_name="s")
    fn = pl.kernel(
        _gather_body,
        mesh=mesh,
        out_type=jax.ShapeDtypeStruct((_TOK, _H), jnp.float32),
        scratch_types=[
            pltpu.VMEM((_NCHUNK, _CH), jnp.int32),
            pltpu.VMEM((_CH, _H), jnp.float32),
            pltpu.SemaphoreType.DMA,
        ],
    )
    return fn(t2, gidx2)


def kernel(time_features, month_table, week_table, holiday_table,
           W_week, b_week, W_holiday, b_holiday, pe):
    tf = time_features.astype(jnp.int32)
    m2 = tf[..., 0].reshape(_TOK // 128, 128)
    w2 = tf[..., 1].reshape(_TOK // 128, 128)
    h2 = tf[..., 2].reshape(_TOK // 128, 128)
    pe2 = pe[0, :_L, :]
    t3 = _build_table(month_table, week_table, holiday_table,
                      W_week, b_week.reshape(1, _H),
                      W_holiday, b_holiday.reshape(1, _H), pe2)
    t2 = t3.reshape(_L * _NCOMBO, _H)
    gidx2 = _build_idx(m2, w2, h2)
    out2 = _sc_gather(t2, gidx2)
    return out2.reshape(_B, _L, _H)
